# baseline (speedup 1.0000x reference)
import functools

import jax
import jax.numpy as jnp
from jax import lax
from jax.experimental import pallas as pl
from jax.experimental.pallas import tpu as pltpu
from jax.experimental.pallas import tpu_sc as plsc

TOPK = 2
CAPF = 1.25
R = 256


def _sc_mesh():
    return plsc.VectorSubcoreMesh(core_axis_name="c", subcore_axis_name="s",
                                  num_cores=2, num_subcores=16)


_SC_PARAMS = pltpu.CompilerParams(needs_layout_passes=False)


def _router_body(E, CAP, S, TB, x_ref, wr_ref, se_ref, so_ref, we_ref, wo_ref,
                 hist_ref):
    b = pl.program_id(0)

    @pl.when(b == 0)
    def _():
        hist_ref[...] = jnp.zeros_like(hist_ref)

    logits = jnp.dot(x_ref[...].astype(jnp.bfloat16),
                     wr_ref[...].astype(jnp.bfloat16),
                     preferred_element_type=jnp.float32)
    iota = lax.broadcasted_iota(jnp.int32, (TB, E), 1)
    m1 = jnp.max(logits, axis=1, keepdims=True)
    a1 = jnp.min(jnp.where(logits >= m1, iota, E), axis=1, keepdims=True)
    masked = jnp.where(iota == a1, -1e30, logits)
    m2 = jnp.max(masked, axis=1, keepdims=True)
    a2 = jnp.min(jnp.where(masked >= m2, iota, E), axis=1, keepdims=True)
    e2 = jnp.exp(m2 - m1)
    w1 = 1.0 / (1.0 + e2)
    w2 = 1.0 - w1

    oh0 = (iota == a1).astype(jnp.float32)
    oh1 = (iota == a2).astype(jnp.float32)
    ohsum = oh0 + oh1
    cs = ohsum
    sh = 1
    while sh < TB:
        z = jnp.zeros((sh, E), jnp.float32)
        cs = cs + jnp.concatenate([z, cs[:TB - sh]], axis=0)
        sh *= 2
    aex = cs - ohsum + hist_ref[...]
    rank0 = jnp.sum(aex * oh0, axis=1, keepdims=True)
    rank1 = jnp.sum(aex * oh1, axis=1, keepdims=True)
    hist_ref[...] = hist_ref[...] + jnp.sum(ohsum, axis=0, keepdims=True)

    slot0 = a1.astype(jnp.float32) * float(CAP) + rank0
    slot1 = a2.astype(jnp.float32) * float(CAP) + rank1
    v0 = rank0 < float(CAP)
    v1 = rank1 < float(CAP)
    se_ref[0] = jnp.where(v0, slot0, float(S)).astype(jnp.int32)
    so_ref[0] = jnp.where(v1, slot1, float(S)).astype(jnp.int32)
    we_ref[0] = jnp.where(v0, w1, 0.0)
    wo_ref[0] = jnp.where(v1, w2, 0.0)


def _router_call(xf, W_router, E, CAP, S, TB):
    T, D = xf.shape
    nb = T // TB
    outs = pl.pallas_call(
        functools.partial(_router_body, E, CAP, S, TB),
        grid=(nb,),
        in_specs=[
            pl.BlockSpec((TB, D), lambda b: (b, 0)),
            pl.BlockSpec((D, E), lambda b: (0, 0)),
        ],
        out_specs=[pl.BlockSpec((1, TB, 1), lambda b: (b, 0, 0))] * 4,
        out_shape=[
            jax.ShapeDtypeStruct((nb, TB, 1), jnp.int32),
            jax.ShapeDtypeStruct((nb, TB, 1), jnp.int32),
            jax.ShapeDtypeStruct((nb, TB, 1), jnp.float32),
            jax.ShapeDtypeStruct((nb, TB, 1), jnp.float32),
        ],
        scratch_shapes=[pltpu.VMEM((1, E), jnp.float32)],
    )(xf, W_router)
    T_ = nb * TB
    return tuple(o.reshape(T_) for o in outs)


def _concat_body(D, x_ref, c_ref, o_ref):
    RT = x_ref.shape[0]
    s1 = D // 128
    o_ref[:, :s1, :] = x_ref[...].reshape(RT, s1, 128)
    s2 = c_ref.shape[1] // 128
    o_ref[:, s1:, :] = c_ref[...].reshape(RT, s2, 128)


def _concat_call(xf, cf):
    T, D = xf.shape
    DC = cf.shape[1]
    RT = 256
    SL = (D + DC) // 128
    return pl.pallas_call(
        functools.partial(_concat_body, D),
        grid=(T // RT,),
        in_specs=[
            pl.BlockSpec((RT, D), lambda i: (i, 0)),
            pl.BlockSpec((RT, DC), lambda i: (i, 0)),
        ],
        out_specs=pl.BlockSpec((RT, SL, 128), lambda i: (i, 0, 0)),
        out_shape=jax.ShapeDtypeStruct((T, SL, 128), jnp.float32),
    )(xf, cf)


def _cast_body(*refs):
    half = len(refs) // 2
    for i_ref, o_ref in zip(refs[:half], refs[half:]):
        o_ref[...] = i_ref[...].astype(jnp.bfloat16)


def _cast_call(eW1, eWc, eW2, sW1, sWc, sW2, E):
    D, DFF = sW1.shape
    DC = sWc.shape[0]
    G = 4 * E
    p1 = D // 4
    p2 = DFF // 4
    rs = D // G
    rs2 = DFF // G
    espec = lambda bs: pl.BlockSpec(bs, lambda g: (g // 4, g % 4, 0))
    sspec = lambda bs: pl.BlockSpec(bs, lambda g: (g, 0))
    specs = [
        espec((1, p1, DFF)), espec((1, p1, DFF)), espec((1, p2, D)),
        sspec((rs, DFF)), sspec((rs, DFF)), sspec((rs2, D)),
    ]
    return pl.pallas_call(
        _cast_body,
        grid=(G,),
        in_specs=specs,
        out_specs=specs,
        out_shape=[
            jax.ShapeDtypeStruct(eW1.shape, jnp.bfloat16),
            jax.ShapeDtypeStruct(eWc.shape, jnp.bfloat16),
            jax.ShapeDtypeStruct(eW2.shape, jnp.bfloat16),
            jax.ShapeDtypeStruct(sW1.shape, jnp.bfloat16),
            jax.ShapeDtypeStruct(sWc.shape, jnp.bfloat16),
            jax.ShapeDtypeStruct(sW2.shape, jnp.bfloat16),
        ],
    )(eW1, eWc, eW2, sW1, sWc, sW2)


def _scatter_body(T, S, SP, se_hbm, so_hbm, we_hbm, wo_hbm, tokf_hbm,
                  wslot_hbm, se_v, so_v, we_v, wo_v, tokf_v, wslot_v):
    c = lax.axis_index("c")
    s = lax.axis_index("s")
    wid = s * 2 + c

    @pl.when(wid == 0)
    def _():
        pltpu.sync_copy(se_hbm, se_v)
        pltpu.sync_copy(so_hbm, so_v)
        pltpu.sync_copy(we_hbm, we_v)
        pltpu.sync_copy(wo_hbm, wo_v)

        def init(i, carry):
            off = pl.multiple_of(i * 16, 16)
            tokf_v[pl.ds(off, 16)] = jnp.zeros((16,), jnp.int32)
            wslot_v[pl.ds(off, 16)] = jnp.zeros((16,), jnp.float32)
            return carry

        lax.fori_loop(0, SP // 16, init, 0)

        def scat(i, carry):
            off = pl.multiple_of(i * 16, 16)
            tv = jnp.arange(16, dtype=jnp.int32) + i * 16
            sv = se_v[pl.ds(off, 16)]
            m = sv < S
            plsc.store_scatter(tokf_v, [sv], tv, mask=m)
            plsc.store_scatter(wslot_v, [sv], we_v[pl.ds(off, 16)], mask=m)
            sv2 = so_v[pl.ds(off, 16)]
            m2 = sv2 < S
            plsc.store_scatter(tokf_v, [sv2], tv, mask=m2)
            plsc.store_scatter(wslot_v, [sv2], wo_v[pl.ds(off, 16)], mask=m2)
            return carry

        lax.fori_loop(0, T // 16, scat, 0)
        pltpu.sync_copy(tokf_v, tokf_hbm)
        pltpu.sync_copy(wslot_v, wslot_hbm)


def _scatter_call(se, so, we, wo, T, S, SP):
    return pl.kernel(
        functools.partial(_scatter_body, T, S, SP),
        out_type=[
            jax.ShapeDtypeStruct((SP,), jnp.int32),
            jax.ShapeDtypeStruct((SP,), jnp.float32),
        ],
        mesh=_sc_mesh(),
        compiler_params=_SC_PARAMS,
        scratch_types=[
            pltpu.VMEM((T,), jnp.int32),
            pltpu.VMEM((T,), jnp.int32),
            pltpu.VMEM((T,), jnp.float32),
            pltpu.VMEM((T,), jnp.float32),
            pltpu.VMEM((SP,), jnp.int32),
            pltpu.VMEM((SP,), jnp.float32),
        ],
    )(se, so, we, wo)


def _gather_body(SP, DX, NW, CH, tokf_hbm, xc_hbm, gxc_hbm,
                 idx_v, r0, r1, s0, s1):
    c = lax.axis_index("c")
    s = lax.axis_index("s")
    wid = s * 2 + c
    per = SP // NW
    base = pl.multiple_of(wid * per, 8)
    pltpu.sync_copy(tokf_hbm.at[pl.ds(base, per)], idx_v)
    nch = per // CH
    tail = per % CH
    bufs = ((r0, s0), (r1, s1))

    def fire(j, buf, sem):
        idx = idx_v.at[pl.ds(pl.multiple_of(j * CH, CH), CH)]
        pltpu.async_copy(xc_hbm.at[idx], buf, sem)

    def drain_out(j, buf, sem):
        pltpu.make_async_copy(xc_hbm.at[pl.ds(0, CH)], buf, sem).wait()
        pltpu.sync_copy(buf.reshape(CH, DX),
                        gxc_hbm.at[pl.ds(base + j * CH, CH)])

    fire(0, *bufs[0])

    def pair(g, carry):
        j0 = 2 * g
        j1 = 2 * g + 1

        @pl.when(j1 < nch)
        def _():
            fire(j1, *bufs[1])

        drain_out(j0, *bufs[0])

        @pl.when(j1 + 1 < nch)
        def _():
            fire(j1 + 1, *bufs[0])

        @pl.when(j1 < nch)
        def _():
            drain_out(j1, *bufs[1])

        return carry

    lax.fori_loop(0, (nch + 1) // 2, pair, 0)
    if tail:
        toff = nch * CH
        idx = idx_v.at[pl.ds(toff, tail)]
        r_t = r0.at[pl.ds(0, tail)]
        pltpu.async_copy(xc_hbm.at[idx], r_t, s0).wait()
        pltpu.sync_copy(r0.reshape(CH, DX).at[pl.ds(0, tail)],
                        gxc_hbm.at[pl.ds(base + toff, tail)])


def _gather_call(tokf, xc3, SP, CH):
    T, SL, L = xc3.shape
    DX = SL * L
    NW = 32
    return pl.kernel(
        functools.partial(_gather_body, SP, DX, NW, CH),
        out_type=jax.ShapeDtypeStruct((SP, DX), jnp.float32),
        mesh=_sc_mesh(),
        compiler_params=_SC_PARAMS,
        scratch_types=[
            pltpu.VMEM((SP // NW,), jnp.int32),
            pltpu.VMEM((CH, SL, L), jnp.float32),
            pltpu.VMEM((CH, SL, L), jnp.float32),
            pltpu.SemaphoreType.DMA,
            pltpu.SemaphoreType.DMA,
        ],
    )(tokf, xc3)


def _ffn_body(TPE, D, x_ref, w1_ref, wc_ref, b1_ref, w2_ref, b2_ref,
              ws_ref, o_ref):
    xb = x_ref[:, :D].astype(jnp.bfloat16)
    cb = x_ref[:, D:].astype(jnp.bfloat16)
    h = jnp.dot(xb, w1_ref[0], preferred_element_type=jnp.float32)
    h = h + jnp.dot(cb, wc_ref[0], preferred_element_type=jnp.float32)
    h = h + b1_ref[0]
    h = jax.nn.gelu(h)
    o = jnp.dot(h.astype(jnp.bfloat16), w2_ref[0],
                preferred_element_type=jnp.float32)
    o = o + b2_ref[0]
    o_ref[...] = o * ws_ref[0]


def _ffn_call(gxc, eW1b, eWcb, eb1, eW2b, eb2, wslot, E, CAP, SP):
    DX = gxc.shape[1]
    DFF = eW1b.shape[2]
    D = eW1b.shape[1]
    DC = DX - D
    TPE = CAP // R
    ntiles = SP // R
    eidx = lambda i: (jnp.minimum(i // TPE, E - 1), 0, 0)
    return pl.pallas_call(
        functools.partial(_ffn_body, TPE, D),
        grid=(ntiles,),
        in_specs=[
            pl.BlockSpec((R, DX), lambda i: (i, 0)),
            pl.BlockSpec((1, D, DFF), eidx),
            pl.BlockSpec((1, DC, DFF), eidx),
            pl.BlockSpec((1, 1, DFF), eidx),
            pl.BlockSpec((1, DFF, D), eidx),
            pl.BlockSpec((1, 1, D), eidx),
            pl.BlockSpec((1, R, 1), lambda i: (i, 0, 0)),
        ],
        out_specs=pl.BlockSpec((R, D), lambda i: (i, 0)),
        out_shape=jax.ShapeDtypeStruct((SP, D), jnp.float32),
    )(gxc, eW1b, eWcb, eb1.reshape(E, 1, DFF), eW2b, eb2.reshape(E, 1, D),
      wslot.reshape(SP // R, R, 1))


def _pair_body(T, NW, CH, se_hbm, so_hbm, xo_hbm, re_hbm, ro_hbm,
               ie_v, io_v, rows_v, sem):
    c = lax.axis_index("c")
    s = lax.axis_index("s")
    wid = s * 2 + c
    per = T // NW
    base = pl.multiple_of(wid * per, 8)
    pltpu.sync_copy(se_hbm.at[pl.ds(base, per)], ie_v)
    pltpu.sync_copy(so_hbm.at[pl.ds(base, per)], io_v)

    def chunk(j, carry):
        off = pl.multiple_of(j * CH, CH)
        pltpu.async_copy(xo_hbm.at[ie_v.at[pl.ds(off, CH)]], rows_v, sem).wait()
        pltpu.sync_copy(rows_v, re_hbm.at[pl.ds(base + off, CH)])
        pltpu.async_copy(xo_hbm.at[io_v.at[pl.ds(off, CH)]], rows_v, sem).wait()
        pltpu.sync_copy(rows_v, ro_hbm.at[pl.ds(base + off, CH)])
        return carry

    lax.fori_loop(0, per // CH, chunk, 0)


def _pair_call(se, so, xo_w, T, CH):
    D = xo_w.shape[1]
    NW = 32
    return pl.kernel(
        functools.partial(_pair_body, T, NW, CH),
        out_type=[
            jax.ShapeDtypeStruct((T, D), jnp.float32),
            jax.ShapeDtypeStruct((T, D), jnp.float32),
        ],
        mesh=_sc_mesh(),
        compiler_params=_SC_PARAMS,
        scratch_types=[
            pltpu.VMEM((T // NW,), jnp.int32),
            pltpu.VMEM((T // NW,), jnp.int32),
            pltpu.VMEM((CH, D), jnp.float32),
            pltpu.SemaphoreType.DMA,
        ],
    )(se, so, xo_w)


def _shared_body(x_ref, c_ref, w1_ref, wc_ref, b1_ref, w2_ref, b2_ref,
                 m_ref, re_ref, ro_ref, o_ref):
    xb = x_ref[...].astype(jnp.bfloat16)
    cb = c_ref[...].astype(jnp.bfloat16)
    h = jnp.dot(xb, w1_ref[...], preferred_element_type=jnp.float32)
    h = h + jnp.dot(cb, wc_ref[...], preferred_element_type=jnp.float32)
    h = h + b1_ref[...]
    h = jax.nn.gelu(h)
    sh = jnp.dot(h.astype(jnp.bfloat16), w2_ref[...],
                 preferred_element_type=jnp.float32)
    sh = sh + b2_ref[...]
    o_ref[...] = m_ref[0] * (sh + 2.0 * (re_ref[...] + ro_ref[...])) / 3.0


def _shared_call(xf, cf, sW1b, sWcb, sb1, sW2b, sb2, mf, rE, rO):
    T, D = xf.shape
    DC = cf.shape[1]
    DFF = sW1b.shape[1]
    nt = T // R
    return pl.pallas_call(
        _shared_body,
        grid=(nt,),
        in_specs=[
            pl.BlockSpec((R, D), lambda i: (i, 0)),
            pl.BlockSpec((R, DC), lambda i: (i, 0)),
            pl.BlockSpec((D, DFF), lambda i: (0, 0)),
            pl.BlockSpec((DC, DFF), lambda i: (0, 0)),
            pl.BlockSpec((1, DFF), lambda i: (0, 0)),
            pl.BlockSpec((DFF, D), lambda i: (0, 0)),
            pl.BlockSpec((1, D), lambda i: (0, 0)),
            pl.BlockSpec((1, R, 1), lambda i: (i, 0, 0)),
            pl.BlockSpec((R, D), lambda i: (i, 0)),
            pl.BlockSpec((R, D), lambda i: (i, 0)),
        ],
        out_specs=pl.BlockSpec((R, D), lambda i: (i, 0)),
        out_shape=jax.ShapeDtypeStruct((T, D), jnp.float32),
    )(xf, cf, sW1b, sWcb, sb1.reshape(1, DFF), sW2b, sb2.reshape(1, D),
      mf.reshape(nt, R, 1), rE, rO)


def kernel(x, cond, mask, W_router, sW1, sWc, sb1, sW2, sb2, eW1, eWc, eb1,
           eW2, eb2):
    b, n, d = x.shape
    T = b * n
    E = W_router.shape[1]
    dc = cond.shape[2]
    CAP = max(int(CAPF * TOPK * T / E), 1)
    S = E * CAP
    SP = S + R

    xf = x.reshape(T, d)
    cf = cond.reshape(T, dc)
    mf = mask.reshape(T)

    se, so, we, wo = _router_call(xf, W_router, E, CAP, S, TB=1024)
    xc = _concat_call(xf, cf)
    tokf, wslot = _scatter_call(se, so, we, wo, T, S, SP)
    eW1b, eWcb, eW2b, sW1b, sWcb, sW2b = _cast_call(eW1, eWc, eW2, sW1, sWc,
                                                    sW2, E)
    gxc = _gather_call(tokf, xc, SP, CH=24)
    xo_w = _ffn_call(gxc, eW1b, eWcb, eb1, eW2b, eb2, wslot, E, CAP, SP)
    rE, rO = _pair_call(se, so, xo_w, T, CH=32)
    out = _shared_call(xf, cf, sW1b, sWcb, sb1, sW2b, sb2, mf, rE, rO)
    return out.reshape(b, n, d)


# speedup vs baseline: 1.1946x; 1.1946x over previous
import functools

import jax
import jax.numpy as jnp
from jax import lax
from jax.experimental import pallas as pl
from jax.experimental.pallas import tpu as pltpu
from jax.experimental.pallas import tpu_sc as plsc

TOPK = 2
CAPF = 1.25
R = 256


def _sc_mesh():
    return plsc.VectorSubcoreMesh(core_axis_name="c", subcore_axis_name="s",
                                  num_cores=2, num_subcores=16)


_SC_PARAMS = pltpu.CompilerParams(needs_layout_passes=False)


def _router_body(E, CAP, S, TB, x_ref, wr_ref, se_ref, so_ref, we_ref, wo_ref,
                 hist_ref):
    b = pl.program_id(0)

    @pl.when(b == 0)
    def _():
        hist_ref[...] = jnp.zeros_like(hist_ref)

    logits = jnp.dot(x_ref[...].astype(jnp.bfloat16),
                     wr_ref[...].astype(jnp.bfloat16),
                     preferred_element_type=jnp.float32)
    iota = lax.broadcasted_iota(jnp.int32, (TB, E), 1)
    m1 = jnp.max(logits, axis=1, keepdims=True)
    a1 = jnp.min(jnp.where(logits >= m1, iota, E), axis=1, keepdims=True)
    masked = jnp.where(iota == a1, -1e30, logits)
    m2 = jnp.max(masked, axis=1, keepdims=True)
    a2 = jnp.min(jnp.where(masked >= m2, iota, E), axis=1, keepdims=True)
    e2 = jnp.exp(m2 - m1)
    w1 = 1.0 / (1.0 + e2)
    w2 = 1.0 - w1

    oh0 = (iota == a1).astype(jnp.float32)
    oh1 = (iota == a2).astype(jnp.float32)
    ohsum = oh0 + oh1
    cs = ohsum
    sh = 1
    while sh < TB:
        z = jnp.zeros((sh, E), jnp.float32)
        cs = cs + jnp.concatenate([z, cs[:TB - sh]], axis=0)
        sh *= 2
    aex = cs - ohsum + hist_ref[...]
    rank0 = jnp.sum(aex * oh0, axis=1, keepdims=True)
    rank1 = jnp.sum(aex * oh1, axis=1, keepdims=True)
    hist_ref[...] = hist_ref[...] + jnp.sum(ohsum, axis=0, keepdims=True)

    slot0 = a1.astype(jnp.float32) * float(CAP) + rank0
    slot1 = a2.astype(jnp.float32) * float(CAP) + rank1
    v0 = rank0 < float(CAP)
    v1 = rank1 < float(CAP)
    se_ref[0] = jnp.where(v0, slot0, float(S)).astype(jnp.int32)
    so_ref[0] = jnp.where(v1, slot1, float(S)).astype(jnp.int32)
    we_ref[0] = jnp.where(v0, w1, 0.0)
    wo_ref[0] = jnp.where(v1, w2, 0.0)


def _router_call(xf, W_router, E, CAP, S, TB):
    T, D = xf.shape
    nb = T // TB
    outs = pl.pallas_call(
        functools.partial(_router_body, E, CAP, S, TB),
        grid=(nb,),
        in_specs=[
            pl.BlockSpec((TB, D), lambda b: (b, 0)),
            pl.BlockSpec((D, E), lambda b: (0, 0)),
        ],
        out_specs=[pl.BlockSpec((1, TB, 1), lambda b: (b, 0, 0))] * 4,
        out_shape=[
            jax.ShapeDtypeStruct((nb, TB, 1), jnp.int32),
            jax.ShapeDtypeStruct((nb, TB, 1), jnp.int32),
            jax.ShapeDtypeStruct((nb, TB, 1), jnp.float32),
            jax.ShapeDtypeStruct((nb, TB, 1), jnp.float32),
        ],
        scratch_shapes=[pltpu.VMEM((1, E), jnp.float32)],
    )(xf, W_router)
    T_ = nb * TB
    return tuple(o.reshape(T_) for o in outs)


def _concat_body(D, x_ref, c_ref, o_ref):
    RT = x_ref.shape[0]
    s1 = D // 128
    o_ref[:, :s1, :] = x_ref[...].reshape(RT, s1, 128)
    s2 = c_ref.shape[1] // 128
    o_ref[:, s1:, :] = c_ref[...].reshape(RT, s2, 128)


def _concat_call(xf, cf):
    T, D = xf.shape
    DC = cf.shape[1]
    RT = 256
    SL = (D + DC) // 128
    return pl.pallas_call(
        functools.partial(_concat_body, D),
        grid=(T // RT,),
        in_specs=[
            pl.BlockSpec((RT, D), lambda i: (i, 0)),
            pl.BlockSpec((RT, DC), lambda i: (i, 0)),
        ],
        out_specs=pl.BlockSpec((RT, SL, 128), lambda i: (i, 0, 0)),
        out_shape=jax.ShapeDtypeStruct((T, SL, 128), jnp.float32),
    )(xf, cf)


def _scatter_body(T, S, SP, se_hbm, so_hbm, we_hbm, wo_hbm, tokf_hbm,
                  wslot_hbm, se_v, so_v, we_v, wo_v, tokf_v, wslot_v):
    c = lax.axis_index("c")
    s = lax.axis_index("s")
    wid = s * 2 + c

    @pl.when(wid == 0)
    def _():
        pltpu.sync_copy(se_hbm, se_v)
        pltpu.sync_copy(so_hbm, so_v)
        pltpu.sync_copy(we_hbm, we_v)
        pltpu.sync_copy(wo_hbm, wo_v)

        def init(i, carry):
            off = pl.multiple_of(i * 16, 16)
            tokf_v[pl.ds(off, 16)] = jnp.zeros((16,), jnp.int32)
            wslot_v[pl.ds(off, 16)] = jnp.zeros((16,), jnp.float32)
            return carry

        lax.fori_loop(0, SP // 16, init, 0)

        def scat(i, carry):
            off = pl.multiple_of(i * 16, 16)
            tv = jnp.arange(16, dtype=jnp.int32) + i * 16
            sv = se_v[pl.ds(off, 16)]
            m = sv < S
            plsc.store_scatter(tokf_v, [sv], tv, mask=m)
            plsc.store_scatter(wslot_v, [sv], we_v[pl.ds(off, 16)], mask=m)
            sv2 = so_v[pl.ds(off, 16)]
            m2 = sv2 < S
            plsc.store_scatter(tokf_v, [sv2], tv, mask=m2)
            plsc.store_scatter(wslot_v, [sv2], wo_v[pl.ds(off, 16)], mask=m2)
            return carry

        lax.fori_loop(0, T // 16, scat, 0)
        pltpu.sync_copy(tokf_v, tokf_hbm)
        pltpu.sync_copy(wslot_v, wslot_hbm)


def _scatter_call(se, so, we, wo, T, S, SP):
    return pl.kernel(
        functools.partial(_scatter_body, T, S, SP),
        out_type=[
            jax.ShapeDtypeStruct((SP,), jnp.int32),
            jax.ShapeDtypeStruct((SP,), jnp.float32),
        ],
        mesh=_sc_mesh(),
        compiler_params=_SC_PARAMS,
        scratch_types=[
            pltpu.VMEM((T,), jnp.int32),
            pltpu.VMEM((T,), jnp.int32),
            pltpu.VMEM((T,), jnp.float32),
            pltpu.VMEM((T,), jnp.float32),
            pltpu.VMEM((SP,), jnp.int32),
            pltpu.VMEM((SP,), jnp.float32),
        ],
    )(se, so, we, wo)


def _gather_body(NQ, QSTART, DX, NW, CH, tokf_hbm, xc_hbm, gxc_hbm,
                 idx_v, r0, r1, s0, s1):
    c = lax.axis_index("c")
    s = lax.axis_index("s")
    wid = s * 2 + c
    per = NQ // NW
    base = pl.multiple_of(wid * per, 8)
    pltpu.sync_copy(tokf_hbm.at[pl.ds(QSTART + base, per)], idx_v)
    nch = per // CH
    tail = per % CH
    bufs = ((r0, s0), (r1, s1))

    def fire(j, buf, sem):
        idx = idx_v.at[pl.ds(pl.multiple_of(j * CH, CH), CH)]
        pltpu.async_copy(xc_hbm.at[idx], buf, sem)

    def drain_out(j, buf, sem):
        pltpu.make_async_copy(xc_hbm.at[pl.ds(0, CH)], buf, sem).wait()
        pltpu.sync_copy(buf.reshape(CH, DX),
                        gxc_hbm.at[pl.ds(base + j * CH, CH)])

    fire(0, *bufs[0])

    def pair(g, carry):
        j0 = 2 * g
        j1 = 2 * g + 1

        @pl.when(j1 < nch)
        def _():
            fire(j1, *bufs[1])

        drain_out(j0, *bufs[0])

        @pl.when(j1 + 1 < nch)
        def _():
            fire(j1 + 1, *bufs[0])

        @pl.when(j1 < nch)
        def _():
            drain_out(j1, *bufs[1])

        return carry

    lax.fori_loop(0, (nch + 1) // 2, pair, 0)
    if tail:
        toff = nch * CH
        idx = idx_v.at[pl.ds(toff, tail)]
        r_t = r0.at[pl.ds(0, tail)]
        pltpu.async_copy(xc_hbm.at[idx], r_t, s0).wait()
        pltpu.sync_copy(r0.reshape(CH, DX).at[pl.ds(0, tail)],
                        gxc_hbm.at[pl.ds(base + toff, tail)])


def _gather_call(tokf, xc3, NQ, QSTART, CH):
    T, SL, L = xc3.shape
    DX = SL * L
    NW = 32
    return pl.kernel(
        functools.partial(_gather_body, NQ, QSTART, DX, NW, CH),
        out_type=jax.ShapeDtypeStruct((NQ, DX), jnp.float32),
        mesh=_sc_mesh(),
        compiler_params=_SC_PARAMS,
        scratch_types=[
            pltpu.VMEM((NQ // NW,), jnp.int32),
            pltpu.VMEM((CH, SL, L), jnp.float32),
            pltpu.VMEM((CH, SL, L), jnp.float32),
            pltpu.SemaphoreType.DMA,
            pltpu.SemaphoreType.DMA,
        ],
    )(tokf, xc3)


def _ffn_body(TPE, D, QT0, has_alias, *refs):
    if has_alias:
        refs = refs[1:]
    x_ref, w1_ref, wc_ref, b1_ref, w2_ref, b2_ref, ws_ref, o_ref = refs
    xb = x_ref[:, :D]
    cb = x_ref[:, D:]
    h = jnp.dot(xb, w1_ref[0], preferred_element_type=jnp.float32)
    h = h + jnp.dot(cb, wc_ref[0], preferred_element_type=jnp.float32)
    h = h + b1_ref[0]
    h = jax.nn.gelu(h)
    o = jnp.dot(h, w2_ref[0], preferred_element_type=jnp.float32)
    o = o + b2_ref[0]
    o_ref[...] = o * ws_ref[0]


def _ffn_call(gxc_q, xo_in, eW1, eWc, eb1, eW2, eb2, wslot, E, CAP, SP, QT0,
              qt):
    DX = gxc_q.shape[1]
    DFF = eW1.shape[2]
    D = eW1.shape[1]
    DC = DX - D
    TPE = CAP // R
    ntiles = gxc_q.shape[0] // R
    eidx = lambda i: (jnp.minimum((i + QT0) // TPE, E - 1), 0, 0)
    has_alias = xo_in is not None
    in_specs = [
        pl.BlockSpec((R, DX), lambda i: (i, 0)),
        pl.BlockSpec((1, D, DFF), eidx),
        pl.BlockSpec((1, DC, DFF), eidx),
        pl.BlockSpec((1, 1, DFF), eidx),
        pl.BlockSpec((1, DFF, D), eidx),
        pl.BlockSpec((1, 1, D), eidx),
        pl.BlockSpec((1, R, 1), lambda i: (i + QT0, 0, 0)),
    ]
    args = [gxc_q, eW1, eWc, eb1.reshape(E, 1, DFF), eW2,
            eb2.reshape(E, 1, D), wslot.reshape(SP // R, R, 1)]
    kwargs = {}
    if has_alias:
        in_specs = [pl.BlockSpec(memory_space=pl.ANY)] + in_specs
        args = [xo_in] + args
        kwargs["input_output_aliases"] = {0: 0}
    return pl.pallas_call(
        functools.partial(_ffn_body, TPE, D, QT0, has_alias),
        grid=(ntiles,),
        in_specs=in_specs,
        out_specs=pl.BlockSpec((R, D), lambda i: (i + QT0, 0)),
        out_shape=jax.ShapeDtypeStruct((SP, D), jnp.float32),
        compiler_params=pltpu.CompilerParams(
            vmem_limit_bytes=112 * 1024 * 1024),
        **kwargs,
    )(*args)


def _pair_body(T, NW, CH, se_hbm, so_hbm, xo_hbm, re_hbm, ro_hbm,
               ie_v, io_v, rows_v, sem):
    c = lax.axis_index("c")
    s = lax.axis_index("s")
    wid = s * 2 + c
    per = T // NW
    base = pl.multiple_of(wid * per, 8)
    pltpu.sync_copy(se_hbm.at[pl.ds(base, per)], ie_v)
    pltpu.sync_copy(so_hbm.at[pl.ds(base, per)], io_v)

    def chunk(j, carry):
        off = pl.multiple_of(j * CH, CH)
        pltpu.async_copy(xo_hbm.at[ie_v.at[pl.ds(off, CH)]], rows_v, sem).wait()
        pltpu.sync_copy(rows_v, re_hbm.at[pl.ds(base + off, CH)])
        pltpu.async_copy(xo_hbm.at[io_v.at[pl.ds(off, CH)]], rows_v, sem).wait()
        pltpu.sync_copy(rows_v, ro_hbm.at[pl.ds(base + off, CH)])
        return carry

    lax.fori_loop(0, per // CH, chunk, 0)


def _pair_call(se, so, xo_w, T, CH):
    D = xo_w.shape[1]
    NW = 32
    return pl.kernel(
        functools.partial(_pair_body, T, NW, CH),
        out_type=[
            jax.ShapeDtypeStruct((T, D), jnp.float32),
            jax.ShapeDtypeStruct((T, D), jnp.float32),
        ],
        mesh=_sc_mesh(),
        compiler_params=_SC_PARAMS,
        scratch_types=[
            pltpu.VMEM((T // NW,), jnp.int32),
            pltpu.VMEM((T // NW,), jnp.int32),
            pltpu.VMEM((CH, D), jnp.float32),
            pltpu.SemaphoreType.DMA,
        ],
    )(se, so, xo_w)


def _shared_body(x_ref, c_ref, w1_ref, wc_ref, b1_ref, w2_ref, b2_ref,
                 m_ref, re_ref, ro_ref, o_ref):
    h = jnp.dot(x_ref[...], w1_ref[...], preferred_element_type=jnp.float32)
    h = h + jnp.dot(c_ref[...], wc_ref[...],
                    preferred_element_type=jnp.float32)
    h = h + b1_ref[...]
    h = jax.nn.gelu(h)
    sh = jnp.dot(h, w2_ref[...], preferred_element_type=jnp.float32)
    sh = sh + b2_ref[...]
    o_ref[...] = m_ref[0] * (sh + 2.0 * (re_ref[...] + ro_ref[...])) / 3.0


def _shared_call(xf, cf, sW1, sWc, sb1, sW2, sb2, mf, rE, rO):
    T, D = xf.shape
    DC = cf.shape[1]
    DFF = sW1.shape[1]
    nt = T // R
    return pl.pallas_call(
        _shared_body,
        grid=(nt,),
        in_specs=[
            pl.BlockSpec((R, D), lambda i: (i, 0)),
            pl.BlockSpec((R, DC), lambda i: (i, 0)),
            pl.BlockSpec((D, DFF), lambda i: (0, 0)),
            pl.BlockSpec((DC, DFF), lambda i: (0, 0)),
            pl.BlockSpec((1, DFF), lambda i: (0, 0)),
            pl.BlockSpec((DFF, D), lambda i: (0, 0)),
            pl.BlockSpec((1, D), lambda i: (0, 0)),
            pl.BlockSpec((1, R, 1), lambda i: (i, 0, 0)),
            pl.BlockSpec((R, D), lambda i: (i, 0)),
            pl.BlockSpec((R, D), lambda i: (i, 0)),
        ],
        out_specs=pl.BlockSpec((R, D), lambda i: (i, 0)),
        out_shape=jax.ShapeDtypeStruct((T, D), jnp.float32),
        compiler_params=pltpu.CompilerParams(
            vmem_limit_bytes=112 * 1024 * 1024),
    )(xf, cf, sW1, sWc, sb1.reshape(1, DFF), sW2, sb2.reshape(1, D),
      mf.reshape(nt, R, 1), rE, rO)


def kernel(x, cond, mask, W_router, sW1, sWc, sb1, sW2, sb2, eW1, eWc, eb1,
           eW2, eb2):
    b, n, d = x.shape
    T = b * n
    E = W_router.shape[1]
    dc = cond.shape[2]
    CAP = max(int(CAPF * TOPK * T / E), 1)
    S = E * CAP
    SP = S + R

    xf = x.reshape(T, d)
    cf = cond.reshape(T, dc)
    mf = mask.reshape(T)

    se, so, we, wo = _router_call(xf, W_router, E, CAP, S, TB=1024)
    xc = _concat_call(xf, cf)
    tokf, wslot = _scatter_call(se, so, we, wo, T, S, SP)

    NPARTS = 4
    EPP = E // NPARTS
    QROWS = EPP * CAP
    xo_w = None
    for q in range(NPARTS):
        qstart = q * QROWS
        nq = QROWS + (SP - NPARTS * QROWS if q == NPARTS - 1 else 0)
        gxc_q = _gather_call(tokf, xc, nq, qstart, CH=24)
        xo_w = _ffn_call(gxc_q, xo_w, eW1, eWc, eb1, eW2, eb2, wslot, E, CAP,
                         SP, qstart // R, q)
    rE, rO = _pair_call(se, so, xo_w, T, CH=32)
    out = _shared_call(xf, cf, sW1, sWc, sb1, sW2, sb2, mf, rE, rO)
    return out.reshape(b, n, d)
